# baseline (speedup 1.0000x reference)
import jax
import jax.numpy as jnp
from jax import lax
from jax.experimental import pallas as pl
from jax.experimental.pallas import tpu as pltpu
from jax.experimental.pallas import tpu_sc as plsc

N = 10000
NPAD = 10240
D_IN = 128
HID = 64
E = 320000
E_TOT = E + N
NW = 32
CH = 64
NCH = 164
EPW = NCH * CH
E_PAD = EPW * NW
ROWS_PT = NPAD // 16
BLK = 1024

f32 = jnp.float32
i32 = jnp.int32


def _dense1_body(x_ref, w_ref, a_ref, h_ref, as_ref, ad_ref, mx_ref):
    i = pl.program_id(0)
    h = jnp.dot(x_ref[...], w_ref[...], preferred_element_type=f32)
    h_ref[...] = h
    a = jnp.dot(h, a_ref[...], preferred_element_type=f32)
    as_ref[...] = a[:, :16]
    ad_ref[...] = a[:, 16:]
    bmax = jnp.broadcast_to(jnp.max(a, axis=0, keepdims=True), (8, 32))

    @pl.when(i == 0)
    def _():
        mx_ref[...] = bmax

    @pl.when(i > 0)
    def _():
        mx_ref[...] = jnp.maximum(mx_ref[...], bmax)


def _dense1(xp, W1i, A1):
    grid = NPAD // BLK
    return pl.pallas_call(
        _dense1_body,
        grid=(grid,),
        in_specs=[
            pl.BlockSpec((BLK, D_IN), lambda i: (i, 0)),
            pl.BlockSpec((D_IN, HID), lambda i: (0, 0)),
            pl.BlockSpec((HID, 32), lambda i: (0, 0)),
        ],
        out_specs=[
            pl.BlockSpec((BLK, HID), lambda i: (i, 0)),
            pl.BlockSpec((BLK, 16), lambda i: (i, 0)),
            pl.BlockSpec((BLK, 16), lambda i: (i, 0)),
            pl.BlockSpec((8, 32), lambda i: (0, 0)),
        ],
        out_shape=[
            jax.ShapeDtypeStruct((NPAD, HID), f32),
            jax.ShapeDtypeStruct((NPAD, 16), f32),
            jax.ShapeDtypeStruct((NPAD, 16), f32),
            jax.ShapeDtypeStruct((8, 32), f32),
        ],
    )(xp, W1i, A1)


def _tile16():
    l = lax.broadcasted_iota(i32, (16, HID), 0)
    hh = lax.broadcasted_iota(i32, (16, HID), 1) % 8
    return (l == hh).astype(f32)


def _elu(x):
    return jnp.where(x > 0, x, jnp.exp(jnp.minimum(x, 0.0)) - 1.0)


def _dense2_body(n0_ref, n1_ref, d0_ref, d1_ref, b1_ref, w_ref, a_ref,
                 h_ref, as_ref, ad_ref, mx_ref):
    i = pl.program_id(0)
    num = n0_ref[...] + n1_ref[...]
    den = d0_ref[...] + d1_ref[...]
    rden = 1.0 / (den + 1e-16)
    rexp = jnp.dot(rden, _tile16(), preferred_element_type=f32)
    g = _elu(num * rexp + b1_ref[...])
    h = jnp.dot(g, w_ref[...], preferred_element_type=f32)
    h_ref[...] = h
    a = jnp.dot(h, a_ref[...], preferred_element_type=f32)
    as_ref[...] = a[:, :16]
    ad_ref[...] = a[:, 16:]
    bmax = jnp.broadcast_to(jnp.max(a, axis=0, keepdims=True), (8, 32))

    @pl.when(i == 0)
    def _():
        mx_ref[...] = bmax

    @pl.when(i > 0)
    def _():
        mx_ref[...] = jnp.maximum(mx_ref[...], bmax)


def _dense2(n0, n1, d0, d1, b1r, W2p, A2):
    grid = NPAD // BLK
    return pl.pallas_call(
        _dense2_body,
        grid=(grid,),
        in_specs=[
            pl.BlockSpec((BLK, HID), lambda i: (i, 0)),
            pl.BlockSpec((BLK, HID), lambda i: (i, 0)),
            pl.BlockSpec((BLK, 16), lambda i: (i, 0)),
            pl.BlockSpec((BLK, 16), lambda i: (i, 0)),
            pl.BlockSpec((1, HID), lambda i: (0, 0)),
            pl.BlockSpec((HID, HID), lambda i: (0, 0)),
            pl.BlockSpec((HID, 32), lambda i: (0, 0)),
        ],
        out_specs=[
            pl.BlockSpec((BLK, HID), lambda i: (i, 0)),
            pl.BlockSpec((BLK, 16), lambda i: (i, 0)),
            pl.BlockSpec((BLK, 16), lambda i: (i, 0)),
            pl.BlockSpec((8, 32), lambda i: (0, 0)),
        ],
        out_shape=[
            jax.ShapeDtypeStruct((NPAD, HID), f32),
            jax.ShapeDtypeStruct((NPAD, 16), f32),
            jax.ShapeDtypeStruct((NPAD, 16), f32),
            jax.ShapeDtypeStruct((8, 32), f32),
        ],
    )(n0, n1, d0, d1, b1r, W2p, A2)


def _final_body(n0_ref, n1_ref, d0_ref, d1_ref, b2_ref, o_ref):
    num = n0_ref[...] + n1_ref[...]
    den = d0_ref[...] + d1_ref[...]
    rden = 1.0 / (den + 1e-16)
    rexp = jnp.dot(rden, _tile16(), preferred_element_type=f32)
    o_ref[...] = num * rexp + b2_ref[...]


def _final(n0, n1, d0, d1, b2r):
    grid = NPAD // BLK
    return pl.pallas_call(
        _final_body,
        grid=(grid,),
        in_specs=[
            pl.BlockSpec((BLK, HID), lambda i: (i, 0)),
            pl.BlockSpec((BLK, HID), lambda i: (i, 0)),
            pl.BlockSpec((BLK, 16), lambda i: (i, 0)),
            pl.BlockSpec((BLK, 16), lambda i: (i, 0)),
            pl.BlockSpec((1, HID), lambda i: (0, 0)),
        ],
        out_specs=pl.BlockSpec((BLK, HID), lambda i: (i, 0)),
        out_shape=jax.ShapeDtypeStruct((NPAD, HID), f32),
    )(n0, n1, d0, d1, b2r)


def _edge_body(src_hbm, dst_hbm, h_hbm, as_hbm, ad_hbm, m_hbm,
               num_hbm, den_hbm,
               srcb, dstb, mb,
               asv0, adv0, hg0, hw0, pv0,
               asv1, adv1, hg1, hw1, pv1,
               z64, z16, num_sh, den_sh,
               gsem0, gsem1, ssem0, ssem1):
    c = lax.axis_index("c")
    s = lax.axis_index("s")
    wid = c * 16 + s
    row0 = s * ROWS_PT

    zero = jnp.zeros((16,), f32)

    def zrow(i, _):
        for j in range(4):
            z64[i, pl.ds(16 * j, 16)] = zero
        z16[i, :] = zero
        return 0

    lax.fori_loop(0, 64, zrow, 0)
    for r in range(ROWS_PT // 64):
        pltpu.sync_copy(z64, num_sh.at[pl.ds(row0 + 64 * r, 64)])
        pltpu.sync_copy(z16, den_sh.at[pl.ds(row0 + 64 * r, 64)])
    plsc.subcore_barrier()

    pltpu.sync_copy(m_hbm, mb)
    M = mb[...]

    slots = [
        (asv0, adv0, hg0, hw0, pv0, gsem0, ssem0),
        (asv1, adv1, hg1, hw1, pv1, gsem1, ssem1),
    ]

    def issue_gathers(ch, b):
        asb, adb, hg, _, _, gsem, _ = slots[b]
        pltpu.async_copy(as_hbm.at[srcb.at[ch]], asb, gsem)
        pltpu.async_copy(ad_hbm.at[dstb.at[ch]], adb, gsem)
        pltpu.async_copy(h_hbm.at[srcb.at[ch]], hg, gsem)

    def wait_gathers(b):
        asb, adb, hg, _, _, gsem, _ = slots[b]
        pltpu.make_async_copy(as_hbm.at[srcb.at[0]], asb, gsem).wait()
        pltpu.make_async_copy(ad_hbm.at[dstb.at[0]], adb, gsem).wait()
        pltpu.make_async_copy(h_hbm.at[srcb.at[0]], hg, gsem).wait()

    def issue_scatters(ch, b):
        _, _, _, hw, pb, _, ssem = slots[b]
        pltpu.async_copy(hw, num_sh.at[dstb.at[ch]], ssem, add=True)
        pltpu.async_copy(pb, den_sh.at[dstb.at[ch]], ssem, add=True)

    def wait_scatters(b):
        _, _, _, hw, pb, _, ssem = slots[b]
        pltpu.make_async_copy(hw, num_sh.at[dstb.at[0]], ssem).wait()
        pltpu.make_async_copy(pb, den_sh.at[dstb.at[0]], ssem).wait()

    def compute(b):
        asb, adb, hg, hw, pb, _, _ = slots[b]

        def edge(i, _):
            u = asb[i] + adb[i]
            e = jnp.where(u >= 0, u, 0.2 * u)
            p = jnp.exp(e - M)
            pb[i] = p
            for j in range(4):
                sl = pl.ds(16 * j, 16)
                hw[i, sl] = hg[i, sl] * p
            return 0

        lax.fori_loop(0, CH, edge, 0)

    pltpu.sync_copy(src_hbm.at[wid], srcb)
    pltpu.sync_copy(dst_hbm.at[wid], dstb)
    issue_gathers(0, 0)

    def pair(k, _):
        for b in (0, 1):
            ch = 2 * k + b
            wait_gathers(b)
            if b == 0:
                issue_gathers(ch + 1, 1)
            else:
                @pl.when(k < NCH // 2 - 1)
                def _():
                    issue_gathers(ch + 1, 0)

            @pl.when(k >= 1)
            def _():
                wait_scatters(b)
            compute(b)
            issue_scatters(ch, b)
        return 0

    lax.fori_loop(0, NCH // 2, pair, 0)
    wait_scatters(0)
    wait_scatters(1)
    plsc.subcore_barrier()

    pltpu.sync_copy(num_sh.at[pl.ds(row0, ROWS_PT)],
                    num_hbm.at[c, pl.ds(row0, ROWS_PT)])
    pltpu.sync_copy(den_sh.at[pl.ds(row0, ROWS_PT)],
                    den_hbm.at[c, pl.ds(row0, ROWS_PT)])


_edge = pl.kernel(
    _edge_body,
    out_type=(
        jax.ShapeDtypeStruct((2, NPAD, HID), f32),
        jax.ShapeDtypeStruct((2, NPAD, 16), f32),
    ),
    mesh=plsc.VectorSubcoreMesh(core_axis_name="c", subcore_axis_name="s",
                                num_cores=2, num_subcores=16),
    scratch_types=[
        pltpu.VMEM((NCH, CH), i32),
        pltpu.VMEM((NCH, CH), i32),
        pltpu.VMEM((16,), f32),
        pltpu.VMEM((CH, 16), f32),
        pltpu.VMEM((CH, 16), f32),
        pltpu.VMEM((CH, HID), f32),
        pltpu.VMEM((CH, HID), f32),
        pltpu.VMEM((CH, 16), f32),
        pltpu.VMEM((CH, 16), f32),
        pltpu.VMEM((CH, 16), f32),
        pltpu.VMEM((CH, HID), f32),
        pltpu.VMEM((CH, HID), f32),
        pltpu.VMEM((CH, 16), f32),
        pltpu.VMEM((64, HID), f32),
        pltpu.VMEM((64, 16), f32),
        pltpu.VMEM_SHARED((NPAD, HID), f32),
        pltpu.VMEM_SHARED((NPAD, 16), f32),
        pltpu.SemaphoreType.DMA,
        pltpu.SemaphoreType.DMA,
        pltpu.SemaphoreType.DMA,
        pltpu.SemaphoreType.DMA,
    ],
    compiler_params=pltpu.CompilerParams(use_tc_tiling_on_sc=False),
)


def _lrelu(x):
    return jnp.where(x >= 0, x, 0.2 * x)


def kernel(x, edge_index, W1, a1s, a1d, b1, W2, a2s, a2d, b2):
    loops = jnp.arange(N, dtype=i32)
    src = jnp.concatenate([
        edge_index[0].astype(i32), loops,
        jnp.zeros((E_PAD - E_TOT,), i32)])
    dst = jnp.concatenate([
        edge_index[1].astype(i32), loops,
        jnp.full((E_PAD - E_TOT,), N, i32)])
    srcg = src.reshape(NW, NCH, CH)
    dstg = dst.reshape(NW, NCH, CH)

    xp = jnp.pad(x, ((0, NPAD - N), (0, 0)))

    perm = (8 * (jnp.arange(HID) % 8) + jnp.arange(HID) // 8)
    W1i = W1[:, perm]
    W2p = W2[perm, :]
    b1r = b1[perm].reshape(1, HID)
    b2r = b2.reshape(1, HID)

    eye8 = jnp.eye(8, dtype=f32)
    A1s = (a1s[0].T[:, :, None] * eye8[None, :, :]).reshape(HID, 8)
    A1d = (a1d[0].T[:, :, None] * eye8[None, :, :]).reshape(HID, 8)
    A1 = jnp.concatenate([A1s, A1s, A1d, A1d], axis=1)
    A2s = a2s[0, 0][:, None] * jnp.ones((1, 16), f32)
    A2d = a2d[0, 0][:, None] * jnp.ones((1, 16), f32)
    A2 = jnp.concatenate([A2s, A2d], axis=1)

    h1, as1, ad1, mx1 = _dense1(xp, W1i, A1)
    m1 = _lrelu(mx1[0, :16] + mx1[0, 16:])
    num1, den1 = _edge(srcg, dstg, h1, as1, ad1, m1)

    h2, as2, ad2, mx2 = _dense2(num1[0], num1[1], den1[0], den1[1],
                                b1r, W2p, A2)
    m2 = _lrelu(mx2[0, :16] + mx2[0, 16:])
    num2, den2 = _edge(srcg, dstg, h2, as2, ad2, m2)

    out = _final(num2[0], num2[1], den2[0], den2[1], b2r)
    return out[:N]


# speedup vs baseline: 1.4542x; 1.4542x over previous
import jax
import jax.numpy as jnp
from jax import lax
from jax.experimental import pallas as pl
from jax.experimental.pallas import tpu as pltpu
from jax.experimental.pallas import tpu_sc as plsc

N = 10000
NPAD = 10240
D_IN = 128
HID = 64
E = 320000
E_TOT = E + N
NW = 32
CH = 64
IB = 18
NBLK = 9
NCH = IB * NBLK
EPW = NCH * CH
E_PAD = EPW * NW
ROWS_PT = NPAD // 16
BLK = 1024

f32 = jnp.float32
i32 = jnp.int32


def _dense1_body(x_ref, w_ref, a_ref, h_ref, as_ref, ad_ref, mx_ref):
    i = pl.program_id(0)
    h = jnp.dot(x_ref[...], w_ref[...], preferred_element_type=f32)
    h_ref[...] = h
    a = jnp.dot(h, a_ref[...], preferred_element_type=f32)
    as_ref[...] = a[:, :HID]
    ad_ref[...] = a[:, HID:]
    bmax = jnp.broadcast_to(jnp.max(a, axis=0, keepdims=True), (8, 2 * HID))

    @pl.when(i == 0)
    def _():
        mx_ref[...] = bmax

    @pl.when(i > 0)
    def _():
        mx_ref[...] = jnp.maximum(mx_ref[...], bmax)


def _dense1(xp, W1, A1):
    grid = NPAD // BLK
    return pl.pallas_call(
        _dense1_body,
        grid=(grid,),
        in_specs=[
            pl.BlockSpec((BLK, D_IN), lambda i: (i, 0)),
            pl.BlockSpec((D_IN, HID), lambda i: (0, 0)),
            pl.BlockSpec((HID, 2 * HID), lambda i: (0, 0)),
        ],
        out_specs=[
            pl.BlockSpec((BLK, HID), lambda i: (i, 0)),
            pl.BlockSpec((BLK, HID), lambda i: (i, 0)),
            pl.BlockSpec((BLK, HID), lambda i: (i, 0)),
            pl.BlockSpec((8, 2 * HID), lambda i: (0, 0)),
        ],
        out_shape=[
            jax.ShapeDtypeStruct((NPAD, HID), f32),
            jax.ShapeDtypeStruct((NPAD, HID), f32),
            jax.ShapeDtypeStruct((NPAD, HID), f32),
            jax.ShapeDtypeStruct((8, 2 * HID), f32),
        ],
    )(xp, W1, A1)


def _elu(x):
    return jnp.where(x > 0, x, jnp.exp(jnp.minimum(x, 0.0)) - 1.0)


def _dense2_body(n0_ref, n1_ref, d0_ref, d1_ref, b1_ref, w_ref, a_ref,
                 h_ref, as_ref, ad_ref, mx_ref):
    i = pl.program_id(0)
    num = n0_ref[...] + n1_ref[...]
    den = d0_ref[...] + d1_ref[...]
    g = _elu(num / (den + 1e-16) + b1_ref[...])
    h = jnp.dot(g, w_ref[...], preferred_element_type=f32)
    h_ref[...] = h
    a = jnp.dot(h, a_ref[...], preferred_element_type=f32)
    as_ref[...] = a[:, :HID]
    ad_ref[...] = a[:, HID:]
    bmax = jnp.broadcast_to(jnp.max(a, axis=0, keepdims=True), (8, 2 * HID))

    @pl.when(i == 0)
    def _():
        mx_ref[...] = bmax

    @pl.when(i > 0)
    def _():
        mx_ref[...] = jnp.maximum(mx_ref[...], bmax)


def _dense2(n0, n1, d0, d1, b1r, W2, A2):
    grid = NPAD // BLK
    return pl.pallas_call(
        _dense2_body,
        grid=(grid,),
        in_specs=[
            pl.BlockSpec((BLK, HID), lambda i: (i, 0)),
            pl.BlockSpec((BLK, HID), lambda i: (i, 0)),
            pl.BlockSpec((BLK, HID), lambda i: (i, 0)),
            pl.BlockSpec((BLK, HID), lambda i: (i, 0)),
            pl.BlockSpec((1, HID), lambda i: (0, 0)),
            pl.BlockSpec((HID, HID), lambda i: (0, 0)),
            pl.BlockSpec((HID, 2 * HID), lambda i: (0, 0)),
        ],
        out_specs=[
            pl.BlockSpec((BLK, HID), lambda i: (i, 0)),
            pl.BlockSpec((BLK, HID), lambda i: (i, 0)),
            pl.BlockSpec((BLK, HID), lambda i: (i, 0)),
            pl.BlockSpec((8, 2 * HID), lambda i: (0, 0)),
        ],
        out_shape=[
            jax.ShapeDtypeStruct((NPAD, HID), f32),
            jax.ShapeDtypeStruct((NPAD, HID), f32),
            jax.ShapeDtypeStruct((NPAD, HID), f32),
            jax.ShapeDtypeStruct((8, 2 * HID), f32),
        ],
    )(n0, n1, d0, d1, b1r, W2, A2)


def _final_body(n0_ref, n1_ref, d0_ref, d1_ref, b2_ref, o_ref):
    num = n0_ref[...] + n1_ref[...]
    den = d0_ref[...] + d1_ref[...]
    o_ref[...] = num / (den + 1e-16) + b2_ref[...]


def _final(n0, n1, d0, d1, b2r):
    grid = NPAD // BLK
    return pl.pallas_call(
        _final_body,
        grid=(grid,),
        in_specs=[
            pl.BlockSpec((BLK, HID), lambda i: (i, 0)),
            pl.BlockSpec((BLK, HID), lambda i: (i, 0)),
            pl.BlockSpec((BLK, HID), lambda i: (i, 0)),
            pl.BlockSpec((BLK, HID), lambda i: (i, 0)),
            pl.BlockSpec((1, HID), lambda i: (0, 0)),
        ],
        out_specs=pl.BlockSpec((BLK, HID), lambda i: (i, 0)),
        out_shape=jax.ShapeDtypeStruct((NPAD, HID), f32),
    )(n0, n1, d0, d1, b2r)


def _edge_body(src_hbm, dst_hbm, h_hbm, as_hbm, ad_hbm, m_hbm,
               num_hbm, den_hbm,
               srcb, dstb, mb,
               asv0, adv0, hg0, hw0, pv0,
               asv1, adv1, hg1, hw1, pv1,
               z64, num_sh, den_sh,
               gsem0, gsem1, ssem0, ssem1):
    c = lax.axis_index("c")
    s = lax.axis_index("s")
    wid = c * 16 + s
    row0 = s * ROWS_PT

    zero = jnp.zeros((16,), f32)

    def zrow(i, _):
        for j in range(4):
            z64[i, pl.ds(16 * j, 16)] = zero
        return 0

    lax.fori_loop(0, 64, zrow, 0)
    for r in range(ROWS_PT // 64):
        pltpu.sync_copy(z64, num_sh.at[pl.ds(row0 + 64 * r, 64)])
        pltpu.sync_copy(z64, den_sh.at[pl.ds(row0 + 64 * r, 64)])
    plsc.subcore_barrier()

    pltpu.sync_copy(m_hbm, mb)
    M = [mb[pl.ds(16 * j, 16)] for j in range(4)]

    slots = [
        (asv0, adv0, hg0, hw0, pv0, gsem0, ssem0),
        (asv1, adv1, hg1, hw1, pv1, gsem1, ssem1),
    ]

    def issue_gathers(ch, b):
        asb, adb, hg, _, _, gsem, _ = slots[b]
        pltpu.async_copy(as_hbm.at[srcb.at[ch]], asb, gsem)
        pltpu.async_copy(ad_hbm.at[dstb.at[ch]], adb, gsem)
        pltpu.async_copy(h_hbm.at[srcb.at[ch]], hg, gsem)

    def wait_gathers(b):
        asb, adb, hg, _, _, gsem, _ = slots[b]
        pltpu.make_async_copy(as_hbm.at[srcb.at[0]], asb, gsem).wait()
        pltpu.make_async_copy(ad_hbm.at[dstb.at[0]], adb, gsem).wait()
        pltpu.make_async_copy(h_hbm.at[srcb.at[0]], hg, gsem).wait()

    def issue_scatters(ch, b):
        _, _, _, hw, pb, _, ssem = slots[b]
        pltpu.async_copy(hw, num_sh.at[dstb.at[ch]], ssem, add=True)
        pltpu.async_copy(pb, den_sh.at[dstb.at[ch]], ssem, add=True)

    def wait_scatters(b):
        _, _, _, hw, pb, _, ssem = slots[b]
        pltpu.make_async_copy(hw, num_sh.at[dstb.at[0]], ssem).wait()
        pltpu.make_async_copy(pb, den_sh.at[dstb.at[0]], ssem).wait()

    def compute(b):
        asb, adb, hg, hw, pb, _, _ = slots[b]

        def edge(i, _):
            for j in range(4):
                sl = pl.ds(16 * j, 16)
                u = asb[i, sl] + adb[i, sl]
                e = jnp.where(u >= 0, u, 0.2 * u)
                p = jnp.exp(e - M[j])
                pb[i, sl] = p
                hw[i, sl] = hg[i, sl] * p
            return 0

        lax.fori_loop(0, CH, edge, 0)

    def block(blk, _):
        pltpu.sync_copy(src_hbm.at[wid, pl.ds(blk * IB, IB)], srcb)
        pltpu.sync_copy(dst_hbm.at[wid, pl.ds(blk * IB, IB)], dstb)
        issue_gathers(0, 0)

        def pair(k, _):
            for b in (0, 1):
                ch = 2 * k + b
                wait_gathers(b)
                if b == 0:
                    issue_gathers(ch + 1, 1)
                else:
                    @pl.when(k < IB // 2 - 1)
                    def _():
                        issue_gathers(ch + 1, 0)

                @pl.when(k >= 1)
                def _():
                    wait_scatters(b)
                compute(b)
                issue_scatters(ch, b)
            return 0

        lax.fori_loop(0, IB // 2, pair, 0)
        wait_scatters(0)
        wait_scatters(1)
        return 0

    lax.fori_loop(0, NBLK, block, 0)
    plsc.subcore_barrier()

    pltpu.sync_copy(num_sh.at[pl.ds(row0, ROWS_PT)],
                    num_hbm.at[c, pl.ds(row0, ROWS_PT)])
    pltpu.sync_copy(den_sh.at[pl.ds(row0, ROWS_PT)],
                    den_hbm.at[c, pl.ds(row0, ROWS_PT)])


_edge = pl.kernel(
    _edge_body,
    out_type=(
        jax.ShapeDtypeStruct((2, NPAD, HID), f32),
        jax.ShapeDtypeStruct((2, NPAD, HID), f32),
    ),
    mesh=plsc.VectorSubcoreMesh(core_axis_name="c", subcore_axis_name="s",
                                num_cores=2, num_subcores=16),
    scratch_types=[
        pltpu.VMEM((IB, CH), i32),
        pltpu.VMEM((IB, CH), i32),
        pltpu.VMEM((HID,), f32),
        pltpu.VMEM((CH, HID), f32),
        pltpu.VMEM((CH, HID), f32),
        pltpu.VMEM((CH, HID), f32),
        pltpu.VMEM((CH, HID), f32),
        pltpu.VMEM((CH, HID), f32),
        pltpu.VMEM((CH, HID), f32),
        pltpu.VMEM((CH, HID), f32),
        pltpu.VMEM((CH, HID), f32),
        pltpu.VMEM((CH, HID), f32),
        pltpu.VMEM((CH, HID), f32),
        pltpu.VMEM((64, HID), f32),
        pltpu.VMEM_SHARED((NPAD, HID), f32),
        pltpu.VMEM_SHARED((NPAD, HID), f32),
        pltpu.SemaphoreType.DMA,
        pltpu.SemaphoreType.DMA,
        pltpu.SemaphoreType.DMA,
        pltpu.SemaphoreType.DMA,
    ],
    compiler_params=pltpu.CompilerParams(use_tc_tiling_on_sc=False),
)


def _lrelu(x):
    return jnp.where(x >= 0, x, 0.2 * x)


def kernel(x, edge_index, W1, a1s, a1d, b1, W2, a2s, a2d, b2):
    loops = jnp.arange(N, dtype=i32)
    src = jnp.concatenate([
        edge_index[0].astype(i32), loops,
        jnp.zeros((E_PAD - E_TOT,), i32)])
    dst = jnp.concatenate([
        edge_index[1].astype(i32), loops,
        jnp.full((E_PAD - E_TOT,), N, i32)])
    srcg = src.reshape(NW, NCH, CH)
    dstg = dst.reshape(NW, NCH, CH)

    xp = jnp.pad(x, ((0, NPAD - N), (0, 0)))

    eye8 = jnp.eye(8, dtype=f32)
    ones8 = jnp.ones((1, 1, 1, 8), f32)
    A1s = (a1s[0][:, :, None, None] * eye8[:, None, :, None] * ones8
           ).reshape(HID, HID)
    A1d = (a1d[0][:, :, None, None] * eye8[:, None, :, None] * ones8
           ).reshape(HID, HID)
    A1 = jnp.concatenate([A1s, A1d], axis=1)
    A2s = a2s[0, 0][:, None] * jnp.ones((1, HID), f32)
    A2d = a2d[0, 0][:, None] * jnp.ones((1, HID), f32)
    A2 = jnp.concatenate([A2s, A2d], axis=1)
    b1r = b1.reshape(1, HID)
    b2r = b2.reshape(1, HID)

    h1, as1, ad1, mx1 = _dense1(xp, W1, A1)
    m1 = _lrelu(mx1[0, :HID] + mx1[0, HID:])
    num1, den1 = _edge(srcg, dstg, h1, as1, ad1, m1)

    h2, as2, ad2, mx2 = _dense2(num1[0], num1[1], den1[0], den1[1],
                                b1r, W2, A2)
    m2 = _lrelu(mx2[0, :HID] + mx2[0, HID:])
    num2, den2 = _edge(srcg, dstg, h2, as2, ad2, m2)

    out = _final(num2[0], num2[1], den2[0], den2[1], b2r)
    return out[:N]
